# baseline (speedup 1.0000x reference)
import functools

import jax
import jax.numpy as jnp
from jax import lax
from jax.experimental import pallas as pl
from jax.experimental.pallas import tpu as pltpu
from jax.experimental.pallas import tpu_sc as plsc

_N = 10000
_E = 320000
_HID = 64
_DIM = 32
_MIN_NORM = 1e-15

_NC = 2
_NS = 16
_NT = _NC * _NS
_C = 125
_EROWS = _E // _C
_SUP = 16
_NPAD = 10240
_NPT = _NPAD // _NS
_W = 128
_BR = 1000


def _artanh(v):
    v = jnp.clip(v, -1.0 + 1e-7, 1.0 - 1e-7)
    return 0.5 * jnp.log((1.0 + v) / (1.0 - v))


def _rnorm(v):
    return jnp.clip(jnp.sqrt(jnp.sum(v * v, axis=-1, keepdims=True)), _MIN_NORM)


def _proj(v, c):
    norm = _rnorm(v)
    maxnorm = (1.0 - 1e-5) / jnp.sqrt(c)
    return jnp.where(norm > maxnorm, v / norm * maxnorm, v)


def _expmap0(u, c):
    sqrt_c = jnp.sqrt(c)
    norm = _rnorm(u)
    return jnp.tanh(sqrt_c * norm) * u / (sqrt_c * norm)


def _logmap0(p, c):
    sqrt_c = jnp.sqrt(c)
    norm = _rnorm(p)
    return _artanh(sqrt_c * norm) * p / (sqrt_c * norm)


def _mobius_add(x, y, c):
    x2 = jnp.sum(x * x, axis=-1, keepdims=True)
    y2 = jnp.sum(y * y, axis=-1, keepdims=True)
    xy = jnp.sum(x * y, axis=-1, keepdims=True)
    num = (1.0 + 2.0 * c * xy + c * y2) * x + (1.0 - c * x2) * y
    den = 1.0 + 2.0 * c * xy + c * c * x2 * y2
    return num / jnp.clip(den, _MIN_NORM)


def _mobius_matvec(W, v, c):
    sqrt_c = jnp.sqrt(c)
    xn = _rnorm(v)
    mx = jnp.dot(v, W, preferred_element_type=jnp.float32)
    mxn = _rnorm(mx)
    return jnp.tanh(mxn / xn * _artanh(sqrt_c * xn)) * mx / (mxn * sqrt_c)


def _pre_agg(x_hyp, W, b, c):
    h = _proj(_mobius_matvec(W, x_hyp, c), c)
    bias = _proj(_expmap0(b, c), c)
    h = _proj(_mobius_add(h, bias, c), c)
    return _logmap0(h, c)


def _tc_stage1(x_ref, w1_ref, b1_ref, c_ref, out_ref):
    c = jnp.abs(c_ref[0, 0]) + 1e-6
    x_hyp = _proj(_expmap0(x_ref[...], c), c)
    t = _pre_agg(x_hyp, w1_ref[...], b1_ref[...], c)
    ones = jnp.ones((_BR, 8), jnp.float32)
    zeros = jnp.zeros((_BR, _W - _HID - 8), jnp.float32)
    out_ref[...] = jnp.concatenate([t, ones, zeros], axis=-1)


def _tc_stage2(sum1_ref, w2_ref, b2_ref, c_ref, t2_ref, deg_ref):
    c = jnp.abs(c_ref[0, 0]) + 1e-6
    s = sum1_ref[0, :, 0:_HID] + sum1_ref[1, :, 0:_HID]
    deg = jnp.clip(
        sum1_ref[0, :, _HID:_HID + 1] + sum1_ref[1, :, _HID:_HID + 1], 1.0)
    h_tan = jax.nn.relu(s / deg)
    z1 = _proj(_expmap0(h_tan, c), c)
    t2 = _pre_agg(z1, w2_ref[...], b2_ref[...], c)
    zeros = jnp.zeros((_BR, _W - _DIM), jnp.float32)
    t2_ref[...] = jnp.concatenate([t2, zeros], axis=-1)
    deg_ref[...] = jnp.broadcast_to(deg, (_BR, 8))


def _tc_stage3(sum2_ref, deg_ref, c_ref, out_ref):
    c = jnp.abs(c_ref[0, 0]) + 1e-6
    s = sum2_ref[0, :, 0:_DIM] + sum2_ref[1, :, 0:_DIM]
    deg = deg_ref[:, 0:1]
    h_tan = s / deg
    z0 = _proj(_expmap0(h_tan, c), c)
    z0_tan = _logmap0(z0, c)
    ones = jnp.ones((_BR, 8), jnp.float32)
    zeros = jnp.zeros((_BR, _W - _DIM - 8), jnp.float32)
    out_ref[...] = jnp.concatenate([z0_tan, ones, zeros], axis=-1)


def _tc_stage4(z0t_ref, vab_ref, vcd_ref, aw_ref, c_ref, out_ref):
    c = jnp.abs(c_ref[0, 0]) + 1e-6
    views = [z0t_ref[:, 0:_DIM]]
    for ref in (vab_ref, vcd_ref):
        for l in range(2):
            s = ref[l, :, 0:_DIM]
            dg = jnp.clip(ref[l, :, _DIM:_DIM + 1], 1.0)
            views.append(s / dg)
    aw = aw_ref[...]
    scores = [jnp.sum(jnp.tanh(v) * aw, axis=-1, keepdims=True) for v in views]
    m = scores[0]
    for s in scores[1:]:
        m = jnp.maximum(m, s)
    es = [jnp.exp(s - m) for s in scores]
    z = es[0]
    for e in es[1:]:
        z = z + e
    comb = (es[0] / z) * views[0]
    for l in range(1, 5):
        comb = comb + (es[l] / z) * views[l]
    out_ref[...] = _proj(_expmap0(comb, c), c)


def _row_spec(shape):
    if shape[-2] > _BR:
        if len(shape) == 2:
            return pl.BlockSpec((_BR, shape[-1]), lambda i: (i, 0))
        return pl.BlockSpec((shape[0], _BR, shape[-1]), lambda i: (0, i, 0))
    nones = (0,) * len(shape)
    return pl.BlockSpec(shape, lambda i, _z=nones: _z)


def _tc_call(body, out_shapes, *args):
    single = not isinstance(out_shapes, list)
    shapes = [out_shapes] if single else out_shapes
    res = pl.pallas_call(
        body,
        grid=(_N // _BR,),
        in_specs=[_row_spec(a.shape) for a in args],
        out_specs=[_row_spec(s) for s in shapes],
        out_shape=[jax.ShapeDtypeStruct(s, jnp.float32) for s in shapes],
    )(*args)
    return res[0] if single else res


_MESH = plsc.VectorSubcoreMesh(core_axis_name="c", subcore_axis_name="s")

_SLAB = 40


def _sc_scratch():
    return (
        [pltpu.VMEM((_SLAB, _C), jnp.int32),
         pltpu.VMEM((_SLAB, _C), jnp.int32)]
        + [pltpu.VMEM((_C, _W), jnp.float32) for _ in range(2)]
        + [pltpu.VMEM_SHARED((_NPAD, _W), jnp.float32)]
        + [pltpu.SemaphoreType.DMA for _ in range(4)]
    )


def _sc_agg_body(table_h, out_h, accum, sidx, didx, bufs, gsems, ssems,
                 zeros_h, cc, ss, load_slab, rows_per_tile):
    my = pl.ds(pl.multiple_of(ss * _NPT, 8), _NPT)
    pltpu.sync_copy(zeros_h, accum.at[my])
    plsc.subcore_barrier()

    def gather(q, b):
        pltpu.async_copy(table_h.at[sidx.at[q]], bufs[b], gsems[b])

    def gather_wait(q, b):
        pltpu.make_async_copy(table_h.at[sidx.at[q]], bufs[b], gsems[b]).wait()

    def scatter(q, b):
        pltpu.async_copy(bufs[b], accum.at[didx.at[q]], ssems[b], add=True)

    def scatter_wait(q, b):
        pltpu.make_async_copy(bufs[b], accum.at[didx.at[q]], ssems[b]).wait()

    for phase in range(rows_per_tile // _SLAB):
        load_slab(phase)
        gather(0, 0)
        gather(1, 1)

        @pl.loop(0, _SLAB // 2 - 1)
        def _(t):
            q = t * 2
            gather_wait(q, 0)
            scatter(q, 0)
            gather_wait(q + 1, 1)
            scatter(q + 1, 1)
            scatter_wait(q, 0)
            gather(q + 2, 0)
            scatter_wait(q + 1, 1)
            gather(q + 3, 1)

        gather_wait(_SLAB - 2, 0)
        scatter(_SLAB - 2, 0)
        gather_wait(_SLAB - 1, 1)
        scatter(_SLAB - 1, 1)
        scatter_wait(_SLAB - 2, 0)
        scatter_wait(_SLAB - 1, 1)

    plsc.subcore_barrier()
    pltpu.sync_copy(accum.at[my], out_h.at[cc, my])


def _sc_agg_half(table, src2d, dst2d, zeros_pad):
    rows_per_tile = _EROWS // _NT

    @functools.partial(
        pl.kernel,
        out_type=jax.ShapeDtypeStruct((_NC, _NPAD, _W), jnp.float32),
        mesh=_MESH,
        scratch_types=_sc_scratch(),
    )
    def k(table_h, src_h, dst_h, zeros_h, out_h, sidx, didx, b0, b1,
          accum, g0, g1, s0, s1):
        cc = lax.axis_index("c")
        ss = lax.axis_index("s")
        base_row = pl.multiple_of((cc * _NS + ss) * rows_per_tile, 8)

        def load_slab(phase):
            row0 = pl.multiple_of(base_row + phase * _SLAB, 8)
            pltpu.sync_copy(src_h.at[pl.ds(row0, _SLAB)], sidx)
            pltpu.sync_copy(dst_h.at[pl.ds(row0, _SLAB)], didx)

        _sc_agg_body(table_h, out_h, accum, sidx, didx, (b0, b1), (g0, g1),
                     (s0, s1), zeros_h, cc, ss, load_slab, rows_per_tile)

    return k(table, src2d, dst2d, zeros_pad)


def _sc_agg_pair(table, src3d, dst3d, zeros_pad):
    rows_per_tile = _EROWS // _NS

    @functools.partial(
        pl.kernel,
        out_type=jax.ShapeDtypeStruct((_NC, _NPAD, _W), jnp.float32),
        mesh=_MESH,
        scratch_types=_sc_scratch(),
    )
    def k(table_h, src_h, dst_h, zeros_h, out_h, sidx, didx, b0, b1,
          accum, g0, g1, s0, s1):
        cc = lax.axis_index("c")
        ss = lax.axis_index("s")
        base_row = pl.multiple_of(ss * rows_per_tile, 8)

        def load_slab(phase):
            row0 = pl.multiple_of(base_row + phase * _SLAB, 8)
            pltpu.sync_copy(src_h.at[cc, pl.ds(row0, _SLAB)], sidx)
            pltpu.sync_copy(dst_h.at[cc, pl.ds(row0, _SLAB)], didx)

        _sc_agg_body(table_h, out_h, accum, sidx, didx, (b0, b1), (g0, g1),
                     (s0, s1), zeros_h, cc, ss, load_slab, rows_per_tile)

    return k(table, src3d, dst3d, zeros_pad)


def kernel(x, adj, k_diffusion_in, k_diffusion_out, k_neighbor_in,
           k_neighbor_out, W1, b1, W2, b2, att_w, c_param):
    c2 = c_param.reshape(1, 1).astype(jnp.float32)
    b1r = b1.reshape(1, -1)
    b2r = b2.reshape(1, -1)
    awr = att_w.reshape(1, -1)

    def split(e):
        e = e.astype(jnp.int32)
        return e[0].reshape(_EROWS, _C), e[1].reshape(_EROWS, _C)

    src_a, dst_a = split(adj)
    view_src, view_dst = [], []
    for e in (k_diffusion_in, k_diffusion_out, k_neighbor_in, k_neighbor_out):
        s, dd = split(e)
        view_src.append(s)
        view_dst.append(dd)
    src_ab = jnp.stack(view_src[0:2])
    dst_ab = jnp.stack(view_dst[0:2])
    src_cd = jnp.stack(view_src[2:4])
    dst_cd = jnp.stack(view_dst[2:4])

    zeros_pad = jnp.zeros((_NPT, _W), jnp.float32)

    table1 = _tc_call(_tc_stage1, (_N, _W), x, W1, b1r, c2)
    sum1 = _sc_agg_half(table1, src_a, dst_a, zeros_pad)
    table2, deg8 = _tc_call(
        _tc_stage2, [(_N, _W), (_N, 8)], sum1, W2, b2r, c2)
    sum2 = _sc_agg_half(table2, src_a, dst_a, zeros_pad)
    z0table = _tc_call(_tc_stage3, (_N, _W), sum2, deg8, c2)
    v_ab = _sc_agg_pair(z0table, src_ab, dst_ab, zeros_pad)
    v_cd = _sc_agg_pair(z0table, src_cd, dst_cd, zeros_pad)
    return _tc_call(_tc_stage4, (_N, _DIM), z0table, v_ab, v_cd, awr, c2)


# speedup vs baseline: 1.2807x; 1.2807x over previous
import functools

import jax
import jax.numpy as jnp
from jax import lax
from jax.experimental import pallas as pl
from jax.experimental.pallas import tpu as pltpu
from jax.experimental.pallas import tpu_sc as plsc

_N = 10000
_E = 320000
_HID = 64
_DIM = 32
_MIN_NORM = 1e-15

_NC = 2
_NS = 16
_NT = _NC * _NS
_C = 125
_EROWS = _E // _C
_SUP = 16
_NPAD = 10240
_NPT = _NPAD // _NS
_W = 128
_BR = 1000


def _artanh(v):
    v = jnp.clip(v, -1.0 + 1e-7, 1.0 - 1e-7)
    return 0.5 * jnp.log((1.0 + v) / (1.0 - v))


def _rnorm(v):
    return jnp.clip(jnp.sqrt(jnp.sum(v * v, axis=-1, keepdims=True)), _MIN_NORM)


def _proj(v, c):
    norm = _rnorm(v)
    maxnorm = (1.0 - 1e-5) / jnp.sqrt(c)
    return jnp.where(norm > maxnorm, v / norm * maxnorm, v)


def _expmap0(u, c):
    sqrt_c = jnp.sqrt(c)
    norm = _rnorm(u)
    return jnp.tanh(sqrt_c * norm) * u / (sqrt_c * norm)


def _logmap0(p, c):
    sqrt_c = jnp.sqrt(c)
    norm = _rnorm(p)
    return _artanh(sqrt_c * norm) * p / (sqrt_c * norm)


def _mobius_add(x, y, c):
    x2 = jnp.sum(x * x, axis=-1, keepdims=True)
    y2 = jnp.sum(y * y, axis=-1, keepdims=True)
    xy = jnp.sum(x * y, axis=-1, keepdims=True)
    num = (1.0 + 2.0 * c * xy + c * y2) * x + (1.0 - c * x2) * y
    den = 1.0 + 2.0 * c * xy + c * c * x2 * y2
    return num / jnp.clip(den, _MIN_NORM)


def _mobius_matvec(W, v, c):
    sqrt_c = jnp.sqrt(c)
    xn = _rnorm(v)
    mx = jnp.dot(v, W, preferred_element_type=jnp.float32)
    mxn = _rnorm(mx)
    return jnp.tanh(mxn / xn * _artanh(sqrt_c * xn)) * mx / (mxn * sqrt_c)


def _pre_agg(x_hyp, W, b, c):
    h = _proj(_mobius_matvec(W, x_hyp, c), c)
    bias = _proj(_expmap0(b, c), c)
    h = _proj(_mobius_add(h, bias, c), c)
    return _logmap0(h, c)


def _tc_stage1(x_ref, w1_ref, b1_ref, c_ref, out_ref):
    c = jnp.abs(c_ref[0, 0]) + 1e-6
    x_hyp = _proj(_expmap0(x_ref[...], c), c)
    t = _pre_agg(x_hyp, w1_ref[...], b1_ref[...], c)
    ones = jnp.ones((_BR, 8), jnp.float32)
    zeros = jnp.zeros((_BR, _W - _HID - 8), jnp.float32)
    out_ref[...] = jnp.concatenate([t, ones, zeros], axis=-1)


def _tc_stage2(sum1_ref, w2_ref, b2_ref, c_ref, t2_ref, deg_ref):
    c = jnp.abs(c_ref[0, 0]) + 1e-6
    s = sum1_ref[0, :, 0:_HID] + sum1_ref[1, :, 0:_HID]
    deg = jnp.clip(
        sum1_ref[0, :, _HID:_HID + 1] + sum1_ref[1, :, _HID:_HID + 1], 1.0)
    h_tan = jax.nn.relu(s / deg)
    z1 = _proj(_expmap0(h_tan, c), c)
    t2 = _pre_agg(z1, w2_ref[...], b2_ref[...], c)
    zeros = jnp.zeros((_BR, _W - _DIM), jnp.float32)
    t2_ref[...] = jnp.concatenate([t2, zeros], axis=-1)
    deg_ref[...] = jnp.broadcast_to(deg, (_BR, 8))


def _tc_stage3(sum2_ref, deg_ref, c_ref, out_ref):
    c = jnp.abs(c_ref[0, 0]) + 1e-6
    s = sum2_ref[0, :, 0:_DIM] + sum2_ref[1, :, 0:_DIM]
    deg = deg_ref[:, 0:1]
    h_tan = s / deg
    z0 = _proj(_expmap0(h_tan, c), c)
    z0_tan = _logmap0(z0, c)
    ones = jnp.ones((_BR, 8), jnp.float32)
    zeros = jnp.zeros((_BR, _W - _DIM - 8), jnp.float32)
    out_ref[...] = jnp.concatenate([z0_tan, ones, zeros], axis=-1)


def _tc_stage4(z0t_ref, vs_ref, aw_ref, c_ref, out_ref):
    c = jnp.abs(c_ref[0, 0]) + 1e-6
    views = [z0t_ref[:, 0:_DIM]]
    for l in range(4):
        s = vs_ref[l, :, 0:_DIM]
        dg = jnp.clip(vs_ref[l, :, _DIM:_DIM + 1], 1.0)
        views.append(s / dg)
    aw = aw_ref[...]
    scores = [jnp.sum(jnp.tanh(v) * aw, axis=-1, keepdims=True) for v in views]
    m = scores[0]
    for s in scores[1:]:
        m = jnp.maximum(m, s)
    es = [jnp.exp(s - m) for s in scores]
    z = es[0]
    for e in es[1:]:
        z = z + e
    comb = (es[0] / z) * views[0]
    for l in range(1, 5):
        comb = comb + (es[l] / z) * views[l]
    out_ref[...] = _proj(_expmap0(comb, c), c)


def _row_spec(shape):
    if shape[-2] > _BR:
        if len(shape) == 2:
            return pl.BlockSpec((_BR, shape[-1]), lambda i: (i, 0))
        return pl.BlockSpec((shape[0], _BR, shape[-1]), lambda i: (0, i, 0))
    nones = (0,) * len(shape)
    return pl.BlockSpec(shape, lambda i, _z=nones: _z)


def _tc_call(body, out_shapes, *args):
    single = not isinstance(out_shapes, list)
    shapes = [out_shapes] if single else out_shapes
    res = pl.pallas_call(
        body,
        grid=(_N // _BR,),
        in_specs=[_row_spec(a.shape) for a in args],
        out_specs=[_row_spec(s) for s in shapes],
        out_shape=[jax.ShapeDtypeStruct(s, jnp.float32) for s in shapes],
    )(*args)
    return res[0] if single else res


_MESH = plsc.VectorSubcoreMesh(core_axis_name="c", subcore_axis_name="s")

_SLAB = 40


def _sc_scratch():
    return (
        [pltpu.VMEM((_SLAB, _C), jnp.int32),
         pltpu.VMEM((_SLAB, _C), jnp.int32)]
        + [pltpu.VMEM((_C, _W), jnp.float32) for _ in range(2)]
        + [pltpu.VMEM_SHARED((_NPAD, _W), jnp.float32)]
        + [pltpu.SemaphoreType.DMA for _ in range(2)]
    )


def _sc_agg_body(table_h, out_h, accum, sidx, didx, bufs, gsems,
                 zeros_h, cc, ss, load_slab, rows_per_tile):
    my = pl.ds(pl.multiple_of(ss * _NPT, 8), _NPT)
    pltpu.sync_copy(zeros_h, accum.at[my])
    plsc.subcore_barrier()

    def gather(q, b):
        pltpu.async_copy(table_h.at[sidx.at[q]], bufs[b], gsems[b])

    def gather_wait(q, b):
        pltpu.make_async_copy(table_h.at[sidx.at[q]], bufs[b], gsems[b]).wait()

    def scatter(q, b):
        pltpu.sync_copy(bufs[b], accum.at[didx.at[q]], add=True)

    for phase in range(rows_per_tile // _SLAB):
        load_slab(phase)
        gather(0, 0)
        gather(1, 1)

        @pl.loop(0, _SLAB // 2 - 1)
        def _(t):
            q = t * 2
            gather_wait(q, 0)
            scatter(q, 0)
            gather(q + 2, 0)
            gather_wait(q + 1, 1)
            scatter(q + 1, 1)
            gather(q + 3, 1)

        gather_wait(_SLAB - 2, 0)
        scatter(_SLAB - 2, 0)
        gather_wait(_SLAB - 1, 1)
        scatter(_SLAB - 1, 1)

    plsc.subcore_barrier()
    pltpu.sync_copy(accum.at[my], out_h.at[cc, my])


def _sc_agg_half(table, src2d, dst2d, zeros_pad):
    rows_per_tile = _EROWS // _NT

    @functools.partial(
        pl.kernel,
        out_type=jax.ShapeDtypeStruct((_NC, _NPAD, _W), jnp.float32),
        mesh=_MESH,
        scratch_types=_sc_scratch(),
    )
    def k(table_h, src_h, dst_h, zeros_h, out_h, sidx, didx, b0, b1,
          accum, g0, g1):
        cc = lax.axis_index("c")
        ss = lax.axis_index("s")
        base_row = pl.multiple_of((cc * _NS + ss) * rows_per_tile, 8)

        def load_slab(phase):
            row0 = pl.multiple_of(base_row + phase * _SLAB, 8)
            pltpu.sync_copy(src_h.at[pl.ds(row0, _SLAB)], sidx)
            pltpu.sync_copy(dst_h.at[pl.ds(row0, _SLAB)], didx)

        _sc_agg_body(table_h, out_h, accum, sidx, didx, (b0, b1), (g0, g1),
                     zeros_h, cc, ss, load_slab, rows_per_tile)

    return k(table, src2d, dst2d, zeros_pad)


def _sc_agg_views(table, src4, dst4, zeros_pad):
    rows_per_tile = _EROWS // _NS

    @functools.partial(
        pl.kernel,
        out_type=jax.ShapeDtypeStruct((4, _NPAD, _W), jnp.float32),
        mesh=_MESH,
        scratch_types=_sc_scratch(),
    )
    def k(table_h, src_h, dst_h, zeros_h, out_h, sidx, didx, b0, b1,
          accum, g0, g1):
        cc = lax.axis_index("c")
        ss = lax.axis_index("s")
        my = pl.ds(pl.multiple_of(ss * _NPT, 8), _NPT)
        base_row = pl.multiple_of(ss * rows_per_tile, 8)

        for vp in range(2):
            lsel = vp * 2 + cc

            def load_slab(phase, lsel=lsel):
                row0 = pl.multiple_of(base_row + phase * _SLAB, 8)
                pltpu.sync_copy(src_h.at[lsel, pl.ds(row0, _SLAB)], sidx)
                pltpu.sync_copy(dst_h.at[lsel, pl.ds(row0, _SLAB)], didx)

            _sc_agg_body(table_h, out_h.at[pl.ds(vp * 2, 2)], accum, sidx,
                         didx, (b0, b1), (g0, g1), zeros_h, cc, ss,
                         load_slab, rows_per_tile)

    return k(table, src4, dst4, zeros_pad)


def kernel(x, adj, k_diffusion_in, k_diffusion_out, k_neighbor_in,
           k_neighbor_out, W1, b1, W2, b2, att_w, c_param):
    c2 = c_param.reshape(1, 1).astype(jnp.float32)
    b1r = b1.reshape(1, -1)
    b2r = b2.reshape(1, -1)
    awr = att_w.reshape(1, -1)

    def split(e):
        e = e.astype(jnp.int32)
        return e[0].reshape(_EROWS, _C), e[1].reshape(_EROWS, _C)

    src_a, dst_a = split(adj)
    view_src, view_dst = [], []
    for e in (k_diffusion_in, k_diffusion_out, k_neighbor_in, k_neighbor_out):
        s, dd = split(e)
        view_src.append(s)
        view_dst.append(dd)
    src_v = jnp.stack(view_src)
    dst_v = jnp.stack(view_dst)

    zeros_pad = jnp.zeros((_NPT, _W), jnp.float32)

    table1 = _tc_call(_tc_stage1, (_N, _W), x, W1, b1r, c2)
    sum1 = _sc_agg_half(table1, src_a, dst_a, zeros_pad)
    table2, deg8 = _tc_call(
        _tc_stage2, [(_N, _W), (_N, 8)], sum1, W2, b2r, c2)
    sum2 = _sc_agg_half(table2, src_a, dst_a, zeros_pad)
    z0table = _tc_call(_tc_stage3, (_N, _W), sum2, deg8, c2)
    vs = _sc_agg_views(z0table, src_v, dst_v, zeros_pad)
    return _tc_call(_tc_stage4, (_N, _DIM), z0table, vs, awr, c2)


# speedup vs baseline: 1.2829x; 1.0018x over previous
import functools

import jax
import jax.numpy as jnp
from jax import lax
from jax.experimental import pallas as pl
from jax.experimental.pallas import tpu as pltpu
from jax.experimental.pallas import tpu_sc as plsc

_N = 10000
_E = 320000
_HID = 64
_DIM = 32
_MIN_NORM = 1e-15

_NC = 2
_NS = 16
_NT = _NC * _NS
_C = 125
_EROWS = _E // _C
_SUP = 16
_NPAD = 10240
_NPT = _NPAD // _NS
_W = 128
_BR = 2000


def _artanh(v):
    v = jnp.clip(v, -1.0 + 1e-7, 1.0 - 1e-7)
    return 0.5 * jnp.log((1.0 + v) / (1.0 - v))


def _rnorm(v):
    return jnp.clip(jnp.sqrt(jnp.sum(v * v, axis=-1, keepdims=True)), _MIN_NORM)


def _proj(v, c):
    norm = _rnorm(v)
    maxnorm = (1.0 - 1e-5) / jnp.sqrt(c)
    return jnp.where(norm > maxnorm, v / norm * maxnorm, v)


def _expmap0(u, c):
    sqrt_c = jnp.sqrt(c)
    norm = _rnorm(u)
    return jnp.tanh(sqrt_c * norm) * u / (sqrt_c * norm)


def _logmap0(p, c):
    sqrt_c = jnp.sqrt(c)
    norm = _rnorm(p)
    return _artanh(sqrt_c * norm) * p / (sqrt_c * norm)


def _mobius_add(x, y, c):
    x2 = jnp.sum(x * x, axis=-1, keepdims=True)
    y2 = jnp.sum(y * y, axis=-1, keepdims=True)
    xy = jnp.sum(x * y, axis=-1, keepdims=True)
    num = (1.0 + 2.0 * c * xy + c * y2) * x + (1.0 - c * x2) * y
    den = 1.0 + 2.0 * c * xy + c * c * x2 * y2
    return num / jnp.clip(den, _MIN_NORM)


def _mobius_matvec(W, v, c):
    sqrt_c = jnp.sqrt(c)
    xn = _rnorm(v)
    mx = jnp.dot(v, W, preferred_element_type=jnp.float32)
    mxn = _rnorm(mx)
    return jnp.tanh(mxn / xn * _artanh(sqrt_c * xn)) * mx / (mxn * sqrt_c)


def _pre_agg(x_hyp, W, b, c):
    h = _proj(_mobius_matvec(W, x_hyp, c), c)
    bias = _proj(_expmap0(b, c), c)
    h = _proj(_mobius_add(h, bias, c), c)
    return _logmap0(h, c)


def _tc_stage1(x_ref, w1_ref, b1_ref, c_ref, out_ref):
    c = jnp.abs(c_ref[0, 0]) + 1e-6
    x_hyp = _proj(_expmap0(x_ref[...], c), c)
    t = _pre_agg(x_hyp, w1_ref[...], b1_ref[...], c)
    ones = jnp.ones((_BR, 8), jnp.float32)
    zeros = jnp.zeros((_BR, _W - _HID - 8), jnp.float32)
    out_ref[...] = jnp.concatenate([t, ones, zeros], axis=-1)


def _tc_stage2(sum1_ref, w2_ref, b2_ref, c_ref, t2_ref, deg_ref):
    c = jnp.abs(c_ref[0, 0]) + 1e-6
    s = sum1_ref[0, :, 0:_HID] + sum1_ref[1, :, 0:_HID]
    deg = jnp.clip(
        sum1_ref[0, :, _HID:_HID + 1] + sum1_ref[1, :, _HID:_HID + 1], 1.0)
    h_tan = jax.nn.relu(s / deg)
    z1 = _proj(_expmap0(h_tan, c), c)
    t2 = _pre_agg(z1, w2_ref[...], b2_ref[...], c)
    zeros = jnp.zeros((_BR, _W - _DIM), jnp.float32)
    t2_ref[...] = jnp.concatenate([t2, zeros], axis=-1)
    deg_ref[...] = jnp.broadcast_to(deg, (_BR, 8))


def _tc_stage3(sum2_ref, deg_ref, c_ref, out_ref):
    c = jnp.abs(c_ref[0, 0]) + 1e-6
    s = sum2_ref[0, :, 0:_DIM] + sum2_ref[1, :, 0:_DIM]
    deg = deg_ref[:, 0:1]
    h_tan = s / deg
    z0 = _proj(_expmap0(h_tan, c), c)
    z0_tan = _logmap0(z0, c)
    ones = jnp.ones((_BR, 8), jnp.float32)
    zeros = jnp.zeros((_BR, _W - _DIM - 8), jnp.float32)
    out_ref[...] = jnp.concatenate([z0_tan, ones, zeros], axis=-1)


def _tc_stage4(z0t_ref, vs_ref, aw_ref, c_ref, out_ref):
    c = jnp.abs(c_ref[0, 0]) + 1e-6
    views = [z0t_ref[:, 0:_DIM]]
    for l in range(4):
        s = vs_ref[l, :, 0:_DIM]
        dg = jnp.clip(vs_ref[l, :, _DIM:_DIM + 1], 1.0)
        views.append(s / dg)
    aw = aw_ref[...]
    scores = [jnp.sum(jnp.tanh(v) * aw, axis=-1, keepdims=True) for v in views]
    m = scores[0]
    for s in scores[1:]:
        m = jnp.maximum(m, s)
    es = [jnp.exp(s - m) for s in scores]
    z = es[0]
    for e in es[1:]:
        z = z + e
    comb = (es[0] / z) * views[0]
    for l in range(1, 5):
        comb = comb + (es[l] / z) * views[l]
    out_ref[...] = _proj(_expmap0(comb, c), c)


def _row_spec(shape):
    if shape[-2] > _BR:
        if len(shape) == 2:
            return pl.BlockSpec((_BR, shape[-1]), lambda i: (i, 0))
        return pl.BlockSpec((shape[0], _BR, shape[-1]), lambda i: (0, i, 0))
    nones = (0,) * len(shape)
    return pl.BlockSpec(shape, lambda i, _z=nones: _z)


def _tc_call(body, out_shapes, *args):
    single = not isinstance(out_shapes, list)
    shapes = [out_shapes] if single else out_shapes
    res = pl.pallas_call(
        body,
        grid=(_N // _BR,),
        in_specs=[_row_spec(a.shape) for a in args],
        out_specs=[_row_spec(s) for s in shapes],
        out_shape=[jax.ShapeDtypeStruct(s, jnp.float32) for s in shapes],
    )(*args)
    return res[0] if single else res


_MESH = plsc.VectorSubcoreMesh(core_axis_name="c", subcore_axis_name="s")

_SLAB = 40


def _sc_scratch():
    return (
        [pltpu.VMEM((_SLAB, _C), jnp.int32),
         pltpu.VMEM((_SLAB, _C), jnp.int32)]
        + [pltpu.VMEM((_C, _W), jnp.float32) for _ in range(2)]
        + [pltpu.VMEM_SHARED((_NPAD, _W), jnp.float32)]
        + [pltpu.SemaphoreType.DMA for _ in range(2)]
    )


def _sc_agg_body(table_h, out_h, accum, sidx, didx, bufs, gsems,
                 zeros_h, cc, ss, load_slab, rows_per_tile):
    my = pl.ds(pl.multiple_of(ss * _NPT, 8), _NPT)
    pltpu.sync_copy(zeros_h, accum.at[my])
    plsc.subcore_barrier()

    def gather(q, b):
        pltpu.async_copy(table_h.at[sidx.at[q]], bufs[b], gsems[b])

    def gather_wait(q, b):
        pltpu.make_async_copy(table_h.at[sidx.at[q]], bufs[b], gsems[b]).wait()

    def scatter(q, b):
        pltpu.sync_copy(bufs[b], accum.at[didx.at[q]], add=True)

    for phase in range(rows_per_tile // _SLAB):
        load_slab(phase)
        gather(0, 0)
        gather(1, 1)

        @pl.loop(0, _SLAB // 2 - 1)
        def _(t):
            q = t * 2
            gather_wait(q, 0)
            scatter(q, 0)
            gather(q + 2, 0)
            gather_wait(q + 1, 1)
            scatter(q + 1, 1)
            gather(q + 3, 1)

        gather_wait(_SLAB - 2, 0)
        scatter(_SLAB - 2, 0)
        gather_wait(_SLAB - 1, 1)
        scatter(_SLAB - 1, 1)

    plsc.subcore_barrier()
    pltpu.sync_copy(accum.at[my], out_h.at[cc, my])


def _sc_agg_half(table, src2d, dst2d, zeros_pad):
    rows_per_tile = _EROWS // _NT

    @functools.partial(
        pl.kernel,
        out_type=jax.ShapeDtypeStruct((_NC, _NPAD, _W), jnp.float32),
        mesh=_MESH,
        scratch_types=_sc_scratch(),
    )
    def k(table_h, src_h, dst_h, zeros_h, out_h, sidx, didx, b0, b1,
          accum, g0, g1):
        cc = lax.axis_index("c")
        ss = lax.axis_index("s")
        base_row = pl.multiple_of((cc * _NS + ss) * rows_per_tile, 8)

        def load_slab(phase):
            row0 = pl.multiple_of(base_row + phase * _SLAB, 8)
            pltpu.sync_copy(src_h.at[pl.ds(row0, _SLAB)], sidx)
            pltpu.sync_copy(dst_h.at[pl.ds(row0, _SLAB)], didx)

        _sc_agg_body(table_h, out_h, accum, sidx, didx, (b0, b1), (g0, g1),
                     zeros_h, cc, ss, load_slab, rows_per_tile)

    return k(table, src2d, dst2d, zeros_pad)


def _sc_agg_views(table, src4, dst4, zeros_pad):
    rows_per_tile = _EROWS // _NS

    @functools.partial(
        pl.kernel,
        out_type=jax.ShapeDtypeStruct((4, _NPAD, _W), jnp.float32),
        mesh=_MESH,
        scratch_types=_sc_scratch(),
    )
    def k(table_h, src_h, dst_h, zeros_h, out_h, sidx, didx, b0, b1,
          accum, g0, g1):
        cc = lax.axis_index("c")
        ss = lax.axis_index("s")
        my = pl.ds(pl.multiple_of(ss * _NPT, 8), _NPT)
        base_row = pl.multiple_of(ss * rows_per_tile, 8)

        for vp in range(2):
            lsel = vp * 2 + cc

            def load_slab(phase, lsel=lsel):
                row0 = pl.multiple_of(base_row + phase * _SLAB, 8)
                pltpu.sync_copy(src_h.at[lsel, pl.ds(row0, _SLAB)], sidx)
                pltpu.sync_copy(dst_h.at[lsel, pl.ds(row0, _SLAB)], didx)

            _sc_agg_body(table_h, out_h.at[pl.ds(vp * 2, 2)], accum, sidx,
                         didx, (b0, b1), (g0, g1), zeros_h, cc, ss,
                         load_slab, rows_per_tile)

    return k(table, src4, dst4, zeros_pad)


def kernel(x, adj, k_diffusion_in, k_diffusion_out, k_neighbor_in,
           k_neighbor_out, W1, b1, W2, b2, att_w, c_param):
    c2 = c_param.reshape(1, 1).astype(jnp.float32)
    b1r = b1.reshape(1, -1)
    b2r = b2.reshape(1, -1)
    awr = att_w.reshape(1, -1)

    def split(e):
        e = e.astype(jnp.int32)
        return e[0].reshape(_EROWS, _C), e[1].reshape(_EROWS, _C)

    src_a, dst_a = split(adj)
    view_src, view_dst = [], []
    for e in (k_diffusion_in, k_diffusion_out, k_neighbor_in, k_neighbor_out):
        s, dd = split(e)
        view_src.append(s)
        view_dst.append(dd)
    src_v = jnp.stack(view_src)
    dst_v = jnp.stack(view_dst)

    zeros_pad = jnp.zeros((_NPT, _W), jnp.float32)

    table1 = _tc_call(_tc_stage1, (_N, _W), x, W1, b1r, c2)
    sum1 = _sc_agg_half(table1, src_a, dst_a, zeros_pad)
    table2, deg8 = _tc_call(
        _tc_stage2, [(_N, _W), (_N, 8)], sum1, W2, b2r, c2)
    sum2 = _sc_agg_half(table2, src_a, dst_a, zeros_pad)
    z0table = _tc_call(_tc_stage3, (_N, _W), sum2, deg8, c2)
    vs = _sc_agg_views(z0table, src_v, dst_v, zeros_pad)
    return _tc_call(_tc_stage4, (_N, _DIM), z0table, vs, awr, c2)


# speedup vs baseline: 1.2840x; 1.0008x over previous
import functools

import jax
import jax.numpy as jnp
from jax import lax
from jax.experimental import pallas as pl
from jax.experimental.pallas import tpu as pltpu
from jax.experimental.pallas import tpu_sc as plsc

_N = 10000
_E = 320000
_HID = 64
_DIM = 32
_MIN_NORM = 1e-15

_NC = 2
_NS = 16
_NT = _NC * _NS
_C = 125
_EROWS = _E // _C
_NPAD = 10240
_NPT = _NPAD // _NS
_W = 128
_BR = 2000


def _artanh(v):
    v = jnp.clip(v, -1.0 + 1e-7, 1.0 - 1e-7)
    return 0.5 * jnp.log((1.0 + v) / (1.0 - v))


def _rnorm(v):
    return jnp.clip(jnp.sqrt(jnp.sum(v * v, axis=-1, keepdims=True)), _MIN_NORM)


def _proj(v, c):
    norm = _rnorm(v)
    maxnorm = (1.0 - 1e-5) / jnp.sqrt(c)
    return jnp.where(norm > maxnorm, v / norm * maxnorm, v)


def _expmap0(u, c):
    sqrt_c = jnp.sqrt(c)
    norm = _rnorm(u)
    return jnp.tanh(sqrt_c * norm) * u / (sqrt_c * norm)


def _logmap0(p, c):
    sqrt_c = jnp.sqrt(c)
    norm = _rnorm(p)
    return _artanh(sqrt_c * norm) * p / (sqrt_c * norm)


def _mobius_add(x, y, c):
    x2 = jnp.sum(x * x, axis=-1, keepdims=True)
    y2 = jnp.sum(y * y, axis=-1, keepdims=True)
    xy = jnp.sum(x * y, axis=-1, keepdims=True)
    num = (1.0 + 2.0 * c * xy + c * y2) * x + (1.0 - c * x2) * y
    den = 1.0 + 2.0 * c * xy + c * c * x2 * y2
    return num / jnp.clip(den, _MIN_NORM)


def _mobius_matvec(W, v, c):
    sqrt_c = jnp.sqrt(c)
    xn = _rnorm(v)
    mx = jnp.dot(v, W, preferred_element_type=jnp.float32)
    mxn = _rnorm(mx)
    return jnp.tanh(mxn / xn * _artanh(sqrt_c * xn)) * mx / (mxn * sqrt_c)


def _pre_agg(x_hyp, W, b, c):
    h = _proj(_mobius_matvec(W, x_hyp, c), c)
    bias = _proj(_expmap0(b, c), c)
    h = _proj(_mobius_add(h, bias, c), c)
    return _logmap0(h, c)


def _tc_stage1(x_ref, w1_ref, b1_ref, c_ref, out_ref):
    c = jnp.abs(c_ref[0, 0]) + 1e-6
    x_hyp = _proj(_expmap0(x_ref[...], c), c)
    t = _pre_agg(x_hyp, w1_ref[...], b1_ref[...], c)
    ones = jnp.ones((_BR, 8), jnp.float32)
    zeros = jnp.zeros((_BR, _W - _HID - 8), jnp.float32)
    out_ref[...] = jnp.concatenate([t, ones, zeros], axis=-1)


def _tc_stage2(sum1_ref, w2_ref, b2_ref, c_ref, t2_ref, deg_ref):
    c = jnp.abs(c_ref[0, 0]) + 1e-6
    s = sum1_ref[0, :, 0:_HID] + sum1_ref[1, :, 0:_HID]
    deg = jnp.clip(
        sum1_ref[0, :, _HID:_HID + 1] + sum1_ref[1, :, _HID:_HID + 1], 1.0)
    h_tan = jax.nn.relu(s / deg)
    z1 = _proj(_expmap0(h_tan, c), c)
    t2 = _pre_agg(z1, w2_ref[...], b2_ref[...], c)
    zeros = jnp.zeros((_BR, _W - _DIM), jnp.float32)
    t2_ref[...] = jnp.concatenate([t2, zeros], axis=-1)
    deg_ref[...] = jnp.broadcast_to(deg, (_BR, 8))


def _tc_stage3(sum2_ref, deg_ref, c_ref, out_ref):
    c = jnp.abs(c_ref[0, 0]) + 1e-6
    s = sum2_ref[0, :, 0:_DIM] + sum2_ref[1, :, 0:_DIM]
    deg = deg_ref[:, 0:1]
    h_tan = s / deg
    z0 = _proj(_expmap0(h_tan, c), c)
    z0_tan = _logmap0(z0, c)
    ones = jnp.ones((_BR, 8), jnp.float32)
    zeros = jnp.zeros((_BR, _W - _DIM - 8), jnp.float32)
    out_ref[...] = jnp.concatenate([z0_tan, ones, zeros], axis=-1)


def _tc_stage4(z0t_ref, vs_ref, aw_ref, c_ref, out_ref):
    c = jnp.abs(c_ref[0, 0]) + 1e-6
    views = [z0t_ref[:, 0:_DIM]]
    for l in range(4):
        s = vs_ref[l, :, 0:_DIM]
        dg = jnp.clip(vs_ref[l, :, _DIM:_DIM + 1], 1.0)
        views.append(s / dg)
    aw = aw_ref[...]
    scores = [jnp.sum(jnp.tanh(v) * aw, axis=-1, keepdims=True) for v in views]
    m = scores[0]
    for s in scores[1:]:
        m = jnp.maximum(m, s)
    es = [jnp.exp(s - m) for s in scores]
    z = es[0]
    for e in es[1:]:
        z = z + e
    comb = (es[0] / z) * views[0]
    for l in range(1, 5):
        comb = comb + (es[l] / z) * views[l]
    out_ref[...] = _proj(_expmap0(comb, c), c)


def _row_spec(shape):
    if shape[-2] > _BR:
        if len(shape) == 2:
            return pl.BlockSpec((_BR, shape[-1]), lambda i: (i, 0))
        return pl.BlockSpec((shape[0], _BR, shape[-1]), lambda i: (0, i, 0))
    nones = (0,) * len(shape)
    return pl.BlockSpec(shape, lambda i, _z=nones: _z)


def _tc_call(body, out_shapes, *args):
    single = not isinstance(out_shapes, list)
    shapes = [out_shapes] if single else out_shapes
    res = pl.pallas_call(
        body,
        grid=(_N // _BR,),
        in_specs=[_row_spec(a.shape) for a in args],
        out_specs=[_row_spec(s) for s in shapes],
        out_shape=[jax.ShapeDtypeStruct(s, jnp.float32) for s in shapes],
    )(*args)
    return res[0] if single else res


_MESH = plsc.VectorSubcoreMesh(core_axis_name="c", subcore_axis_name="s")

_SLAB = 40


def _sc_scratch():
    return (
        [pltpu.VMEM((_SLAB, _C), jnp.int32),
         pltpu.VMEM((_SLAB, _C), jnp.int32)]
        + [pltpu.VMEM((_C, _W), jnp.float32) for _ in range(2)]
        + [pltpu.VMEM_SHARED((_NPAD, _W), jnp.float32)]
        + [pltpu.SemaphoreType.DMA for _ in range(2)]
    )


def _sc_agg_body(table_h, out_h, accum, sidx, didx, bufs, gsems,
                 zeros_h, cc, ss, load_slab, rows_per_tile):
    my = pl.ds(pl.multiple_of(ss * _NPT, 8), _NPT)
    pltpu.sync_copy(zeros_h, accum.at[my])
    plsc.subcore_barrier()

    def gather(q, b):
        pltpu.async_copy(table_h.at[sidx.at[q]], bufs[b], gsems[b])

    def gather_wait(q, b):
        pltpu.make_async_copy(table_h.at[sidx.at[q]], bufs[b], gsems[b]).wait()

    def scatter(q, b):
        pltpu.sync_copy(bufs[b], accum.at[didx.at[q]], add=True)

    for phase in range(rows_per_tile // _SLAB):
        load_slab(phase)
        gather(0, 0)
        gather(1, 1)

        @pl.loop(0, _SLAB // 2 - 1)
        def _(t):
            q = t * 2
            gather_wait(q, 0)
            scatter(q, 0)
            gather(q + 2, 0)
            gather_wait(q + 1, 1)
            scatter(q + 1, 1)
            gather(q + 3, 1)

        gather_wait(_SLAB - 2, 0)
        scatter(_SLAB - 2, 0)
        gather_wait(_SLAB - 1, 1)
        scatter(_SLAB - 1, 1)

    plsc.subcore_barrier()
    pltpu.sync_copy(accum.at[my], out_h.at[cc, my])


def _sc_agg_half(table, src2d, dst2d, zeros_pad):
    rows_per_tile = _EROWS // _NT

    @functools.partial(
        pl.kernel,
        out_type=jax.ShapeDtypeStruct((_NC, _NPAD, _W), jnp.float32),
        mesh=_MESH,
        scratch_types=_sc_scratch(),
    )
    def k(table_h, src_h, dst_h, zeros_h, out_h, sidx, didx, b0, b1,
          accum, g0, g1):
        cc = lax.axis_index("c")
        ss = lax.axis_index("s")
        base_row = pl.multiple_of((cc * _NS + ss) * rows_per_tile, 8)

        def load_slab(phase):
            row0 = pl.multiple_of(base_row + phase * _SLAB, 8)
            pltpu.sync_copy(src_h.at[pl.ds(row0, _SLAB)], sidx)
            pltpu.sync_copy(dst_h.at[pl.ds(row0, _SLAB)], didx)

        _sc_agg_body(table_h, out_h, accum, sidx, didx, (b0, b1), (g0, g1),
                     zeros_h, cc, ss, load_slab, rows_per_tile)

    return k(table, src2d, dst2d, zeros_pad)


def _sc_agg_views(table, src4, dst4, zeros_pad):
    rows_per_tile = _EROWS // _NS

    @functools.partial(
        pl.kernel,
        out_type=jax.ShapeDtypeStruct((4, _NPAD, _W), jnp.float32),
        mesh=_MESH,
        scratch_types=_sc_scratch(),
    )
    def k(table_h, src_h, dst_h, zeros_h, out_h, sidx, didx, b0, b1,
          accum, g0, g1):
        cc = lax.axis_index("c")
        ss = lax.axis_index("s")
        my = pl.ds(pl.multiple_of(ss * _NPT, 8), _NPT)
        base_row = pl.multiple_of(ss * rows_per_tile, 8)

        for vp in range(2):
            lsel = vp * 2 + cc

            def load_slab(phase, lsel=lsel):
                row0 = pl.multiple_of(base_row + phase * _SLAB, 8)
                pltpu.sync_copy(src_h.at[lsel, pl.ds(row0, _SLAB)], sidx)
                pltpu.sync_copy(dst_h.at[lsel, pl.ds(row0, _SLAB)], didx)

            _sc_agg_body(table_h, out_h.at[pl.ds(vp * 2, 2)], accum, sidx,
                         didx, (b0, b1), (g0, g1), zeros_h, cc, ss,
                         load_slab, rows_per_tile)

    return k(table, src4, dst4, zeros_pad)


def kernel(x, adj, k_diffusion_in, k_diffusion_out, k_neighbor_in,
           k_neighbor_out, W1, b1, W2, b2, att_w, c_param):
    c2 = c_param.reshape(1, 1).astype(jnp.float32)
    b1r = b1.reshape(1, -1)
    b2r = b2.reshape(1, -1)
    awr = att_w.reshape(1, -1)

    def split(e):
        e = e.astype(jnp.int32)
        return e[0].reshape(_EROWS, _C), e[1].reshape(_EROWS, _C)

    src_a, dst_a = split(adj)
    view_src, view_dst = [], []
    for e in (k_diffusion_in, k_diffusion_out, k_neighbor_in, k_neighbor_out):
        s, dd = split(e)
        view_src.append(s)
        view_dst.append(dd)
    src_v = jnp.stack(view_src)
    dst_v = jnp.stack(view_dst)

    zeros_pad = jnp.zeros((_NPT, _W), jnp.float32)

    table1 = _tc_call(_tc_stage1, (_N, _W), x, W1, b1r, c2)
    sum1 = _sc_agg_half(table1, src_a, dst_a, zeros_pad)
    table2, deg8 = _tc_call(
        _tc_stage2, [(_N, _W), (_N, 8)], sum1, W2, b2r, c2)
    sum2 = _sc_agg_half(table2, src_a, dst_a, zeros_pad)
    z0table = _tc_call(_tc_stage3, (_N, _W), sum2, deg8, c2)
    vs = _sc_agg_views(z0table, src_v, dst_v, zeros_pad)
    return _tc_call(_tc_stage4, (_N, _DIM), z0table, vs, awr, c2)
